# Initial kernel scaffold; baseline (speedup 1.0000x reference)
import jax
import jax.numpy as jnp
from jax import lax
from jax.experimental import pallas as pl
from jax.experimental.pallas import tpu as pltpu
from jax.experimental.pallas import tpu_sc as plsc

N = 10000
D = 128
EPS = 1e-3

NC = 2
NS = 16
EB = 128

N_PAD = 10240
ZROWS = N_PAD // NS
WROWS = N // NS
DEGW = 16

_mesh = plsc.VectorSubcoreMesh(core_axis_name="c", subcore_axis_name="s")


def _degree_body(dstp_hbm, out_hbm, dst_v, ones_v, zb_v, acc):
    c = lax.axis_index("c")
    s = lax.axis_index("s")
    nb = dst_v.shape[0]
    pltpu.sync_copy(dstp_hbm.at[c, s], dst_v)

    e0 = (lax.iota(jnp.int32, 16) == 0).astype(jnp.float32)
    z16 = jnp.zeros((16,), jnp.float32)

    def fill(i, _):
        ones_v[i, :] = e0
        return _

    lax.fori_loop(0, EB, fill, None)

    def zfill(i, _):
        zb_v[i, :] = z16
        return _

    lax.fori_loop(0, ZROWS, zfill, None)
    pltpu.sync_copy(zb_v, acc.at[pl.ds(s * ZROWS, ZROWS)])
    plsc.subcore_barrier()

    def step(j, _):
        pltpu.sync_copy(ones_v, acc.at[dst_v.at[j]], add=True)
        return _

    lax.fori_loop(0, nb, step, None)
    plsc.subcore_barrier()
    pltpu.sync_copy(acc.at[pl.ds(s * WROWS, WROWS)],
                    out_hbm.at[c, pl.ds(s * WROWS, WROWS)])


def _scatter_body(h_hbm, srcp_hbm, dstp_hbm, out_hbm,
                  src_v, dst_v, buf0, buf1, sem0, sem1, acc):
    c = lax.axis_index("c")
    s = lax.axis_index("s")
    nb = src_v.shape[0]
    pltpu.sync_copy(srcp_hbm.at[c, s], src_v)
    pltpu.sync_copy(dstp_hbm.at[c, s], dst_v)

    z16 = jnp.zeros((16,), jnp.float32)

    def zfill(i, _):
        for k in range(D // 16):
            buf0[i, pl.ds(k * 16, 16)] = z16
        return _

    lax.fori_loop(0, EB, zfill, None)
    for t in range(ZROWS // EB):
        pltpu.sync_copy(buf0, acc.at[pl.ds(s * ZROWS + t * EB, EB)])
    plsc.subcore_barrier()

    def step(j, _):
        pltpu.async_copy(h_hbm.at[src_v.at[j]], buf0, sem0).wait()
        pltpu.sync_copy(buf0, acc.at[dst_v.at[j]], add=True)
        return _

    lax.fori_loop(0, nb, step, None)
    plsc.subcore_barrier()
    pltpu.sync_copy(acc.at[pl.ds(s * WROWS, WROWS)],
                    out_hbm.at[c, pl.ds(s * WROWS, WROWS)])


def _make_sc_kernels(nb):
    deg = pl.kernel(
        _degree_body,
        out_type=jax.ShapeDtypeStruct((NC, N, DEGW), jnp.float32),
        mesh=_mesh,
        scratch_types=[
            pltpu.VMEM((nb, EB), jnp.int32),
            pltpu.VMEM((EB, DEGW), jnp.float32),
            pltpu.VMEM((ZROWS, DEGW), jnp.float32),
            pltpu.VMEM_SHARED((N_PAD, DEGW), jnp.float32),
        ],
    )
    scat = pl.kernel(
        _scatter_body,
        out_type=jax.ShapeDtypeStruct((NC, N, D), jnp.float32),
        mesh=_mesh,
        scratch_types=[
            pltpu.VMEM((nb, EB), jnp.int32),
            pltpu.VMEM((nb, EB), jnp.int32),
            pltpu.VMEM((EB, D), jnp.float32),
            pltpu.VMEM((EB, D), jnp.float32),
            pltpu.SemaphoreType.DMA,
            pltpu.SemaphoreType.DMA,
            pltpu.VMEM_SHARED((N_PAD, D), jnp.float32),
        ],
    )
    return deg, scat


_BN_C = float(1.0 / (1.0 + EPS) ** 0.5)
_RB = 1000


def _premm_body(x_ref, d0_ref, d1_ref, w_ref, o_ref):
    sc = lax.rsqrt(1.0 + d0_ref[:, 0:1] + d1_ref[:, 0:1])
    o_ref[...] = jnp.dot(x_ref[...] * sc, w_ref[...],
                         preferred_element_type=jnp.float32)


def _combmm_body(a0_ref, a1_ref, hh_ref, d0_ref, d1_ref,
                 b_ref, g_ref, be_ref, w_ref, o_ref):
    sc = lax.rsqrt(1.0 + d0_ref[:, 0:1] + d1_ref[:, 0:1])
    agg = sc * (a0_ref[...] + a1_ref[...] + hh_ref[...])
    y = (agg + b_ref[...]) * (g_ref[...] * _BN_C) + be_ref[...]
    h = jnp.maximum(y, 0.0)
    o_ref[...] = jnp.dot(h * sc, w_ref[...],
                         preferred_element_type=jnp.float32)


def _final_body(a0_ref, a1_ref, hh_ref, d0_ref, d1_ref,
                b_ref, g_ref, be_ref, o_ref):
    sc = lax.rsqrt(1.0 + d0_ref[:, 0:1] + d1_ref[:, 0:1])
    agg = sc * (a0_ref[...] + a1_ref[...] + hh_ref[...])
    y = (agg + b_ref[...]) * (g_ref[...] * _BN_C) + be_ref[...]
    o_ref[...] = jnp.maximum(y, 0.0)


def _row_spec(w):
    return pl.BlockSpec((_RB, w), lambda i: (i, 0))


def _full_spec(h, w):
    return pl.BlockSpec((h, w), lambda i: (0, 0))


_GRID = (N // _RB,)

_premm = pl.pallas_call(
    _premm_body,
    grid=_GRID,
    in_specs=[_row_spec(D), _row_spec(DEGW), _row_spec(DEGW), _full_spec(D, D)],
    out_specs=_row_spec(D),
    out_shape=jax.ShapeDtypeStruct((N, D), jnp.float32),
)

_combmm = pl.pallas_call(
    _combmm_body,
    grid=_GRID,
    in_specs=[_row_spec(D), _row_spec(D), _row_spec(D),
              _row_spec(DEGW), _row_spec(DEGW),
              _full_spec(1, D), _full_spec(1, D), _full_spec(1, D),
              _full_spec(D, D)],
    out_specs=_row_spec(D),
    out_shape=jax.ShapeDtypeStruct((N, D), jnp.float32),
)

_final = pl.pallas_call(
    _final_body,
    grid=_GRID,
    in_specs=[_row_spec(D), _row_spec(D), _row_spec(D),
              _row_spec(DEGW), _row_spec(DEGW),
              _full_spec(1, D), _full_spec(1, D), _full_spec(1, D)],
    out_specs=_row_spec(D),
    out_shape=jax.ShapeDtypeStruct((N, D), jnp.float32),
)


def kernel(x, edge_index, W1, b1, g1, be1, W2, b2, g2, be2, W3, b3, g3, be3):
    src = edge_index[0].astype(jnp.int32)
    dst = edge_index[1].astype(jnp.int32)
    e = src.shape[0]
    nb = -(-e // (NC * NS * EB))
    e_pad = NC * NS * nb * EB
    pad = e_pad - e
    srcp = jnp.concatenate([src, jnp.zeros((pad,), jnp.int32)])
    dstp = jnp.concatenate([dst, jnp.full((pad,), N, jnp.int32)])
    srcp = srcp.reshape(NC, NS, nb, EB)
    dstp = dstp.reshape(NC, NS, nb, EB)

    sc_deg, sc_scat = _make_sc_kernels(nb)

    degp = sc_deg(dstp)
    d0, d1 = degp[0], degp[1]
    b1r, g1r, be1r = b1.reshape(1, D), g1.reshape(1, D), be1.reshape(1, D)
    b2r, g2r, be2r = b2.reshape(1, D), g2.reshape(1, D), be2.reshape(1, D)
    b3r, g3r, be3r = b3.reshape(1, D), g3.reshape(1, D), be3.reshape(1, D)

    hh1 = _premm(x, d0, d1, W1)
    p1 = sc_scat(hh1, srcp, dstp)
    hh2 = _combmm(p1[0], p1[1], hh1, d0, d1, b1r, g1r, be1r, W2)
    p2 = sc_scat(hh2, srcp, dstp)
    hh3 = _combmm(p2[0], p2[1], hh2, d0, d1, b2r, g2r, be2r, W3)
    p3 = sc_scat(hh3, srcp, dstp)
    return _final(p3[0], p3[1], hh3, d0, d1, b3r, g3r, be3r)


# speedup vs baseline: 7.0199x; 7.0199x over previous
import jax
import jax.numpy as jnp
from jax import lax
from jax.experimental import pallas as pl
from jax.experimental.pallas import tpu as pltpu
from jax.experimental.pallas import tpu_sc as plsc

N = 10000
D = 128
EPS = 1e-3

NC = 2
NS = 16
EB = 128

N_PAD = 10240
ZROWS = N_PAD // NS
DEGW = 128

_mesh = plsc.VectorSubcoreMesh(core_axis_name="c", subcore_axis_name="s")


def _degree_body(dstp_hbm, out_hbm, dst_v, drow_v, ones_v, zb_v, acc):
    c = lax.axis_index("c")
    s = lax.axis_index("s")
    nb = dst_v.shape[0]
    pltpu.sync_copy(dstp_hbm.at[c, s], dst_v)

    e1 = jnp.full((16,), 1.0, jnp.float32)
    z16 = jnp.zeros((16,), jnp.float32)

    def fill(i, _):
        for k in range(DEGW // 16):
            ones_v[i, pl.ds(k * 16, 16)] = e1
        return _

    lax.fori_loop(0, EB, fill, None)

    def zfill(i, _):
        for k in range(DEGW // 16):
            zb_v[i, pl.ds(k * 16, 16)] = z16
        return _

    lax.fori_loop(0, EB, zfill, None)
    for t in range(ZROWS // EB):
        pltpu.sync_copy(zb_v, acc.at[pl.ds(s * ZROWS + t * EB, EB)])
    plsc.subcore_barrier()

    def step(j, _):
        for k in range(EB // 16):
            drow_v[pl.ds(k * 16, 16)] = dst_v[j, pl.ds(k * 16, 16)]
        pltpu.sync_copy(ones_v, acc.at[drow_v], add=True)
        return _

    lax.fori_loop(0, nb, step, None)
    plsc.subcore_barrier()
    pltpu.sync_copy(acc.at[pl.ds(s * ZROWS, ZROWS)],
                    out_hbm.at[c, pl.ds(s * ZROWS, ZROWS)])


def _scatter_body(h_hbm, srcp_hbm, dstp_hbm, out_hbm,
                  src_v, dst_v, srow_v, drow_v, buf0, buf1, sem0, sem1, acc):
    c = lax.axis_index("c")
    s = lax.axis_index("s")
    nb = src_v.shape[0]
    pltpu.sync_copy(srcp_hbm.at[c, s], src_v)
    pltpu.sync_copy(dstp_hbm.at[c, s], dst_v)

    z16 = jnp.zeros((16,), jnp.float32)

    def zfill(i, _):
        for k in range(D // 16):
            buf0[i, pl.ds(k * 16, 16)] = z16
        return _

    lax.fori_loop(0, EB, zfill, None)
    for t in range(ZROWS // EB):
        pltpu.sync_copy(buf0, acc.at[pl.ds(s * ZROWS + t * EB, EB)])
    plsc.subcore_barrier()

    def step(j, _):
        for k in range(EB // 16):
            srow_v[pl.ds(k * 16, 16)] = src_v[j, pl.ds(k * 16, 16)]
            drow_v[pl.ds(k * 16, 16)] = dst_v[j, pl.ds(k * 16, 16)]
        pltpu.async_copy(h_hbm.at[srow_v], buf0, sem0).wait()
        pltpu.sync_copy(buf0, acc.at[drow_v], add=True)
        return _

    lax.fori_loop(0, nb, step, None)
    plsc.subcore_barrier()
    pltpu.sync_copy(acc.at[pl.ds(s * ZROWS, ZROWS)],
                    out_hbm.at[c, pl.ds(s * ZROWS, ZROWS)])


def _make_sc_kernels(nb):
    deg = pl.kernel(
        _degree_body,
        out_type=jax.ShapeDtypeStruct((NC, N_PAD, DEGW), jnp.float32),
        mesh=_mesh,
        scratch_types=[
            pltpu.VMEM((nb, EB), jnp.int32),
            pltpu.VMEM((EB,), jnp.int32),
            pltpu.VMEM((EB, DEGW), jnp.float32),
            pltpu.VMEM((EB, DEGW), jnp.float32),
            pltpu.VMEM_SHARED((N_PAD, DEGW), jnp.float32),
        ],
    )
    scat = pl.kernel(
        _scatter_body,
        out_type=jax.ShapeDtypeStruct((NC, N_PAD, D), jnp.float32),
        mesh=_mesh,
        scratch_types=[
            pltpu.VMEM((nb, EB), jnp.int32),
            pltpu.VMEM((nb, EB), jnp.int32),
            pltpu.VMEM((EB,), jnp.int32),
            pltpu.VMEM((EB,), jnp.int32),
            pltpu.VMEM((EB, D), jnp.float32),
            pltpu.VMEM((EB, D), jnp.float32),
            pltpu.SemaphoreType.DMA,
            pltpu.SemaphoreType.DMA,
            pltpu.VMEM_SHARED((N_PAD, D), jnp.float32),
        ],
    )
    return deg, scat


_BN_C = float(1.0 / (1.0 + EPS) ** 0.5)
_RB = 1000


def _premm_body(x_ref, d_ref, w_ref, o_ref):
    sc = lax.rsqrt(1.0 + d_ref[0, :, 0:1] + d_ref[1, :, 0:1])
    o_ref[...] = jnp.dot(x_ref[...] * sc, w_ref[...],
                         preferred_element_type=jnp.float32)


def _combmm_body(p_ref, hh_ref, d_ref, b_ref, g_ref, be_ref, w_ref, o_ref):
    sc = lax.rsqrt(1.0 + d_ref[0, :, 0:1] + d_ref[1, :, 0:1])
    agg = sc * (p_ref[0] + p_ref[1] + hh_ref[...])
    y = (agg + b_ref[...]) * (g_ref[...] * _BN_C) + be_ref[...]
    h = jnp.maximum(y, 0.0)
    o_ref[...] = jnp.dot(h * sc, w_ref[...],
                         preferred_element_type=jnp.float32)


def _final_body(p_ref, hh_ref, d_ref, b_ref, g_ref, be_ref, o_ref):
    sc = lax.rsqrt(1.0 + d_ref[0, :, 0:1] + d_ref[1, :, 0:1])
    agg = sc * (p_ref[0] + p_ref[1] + hh_ref[...])
    y = (agg + b_ref[...]) * (g_ref[...] * _BN_C) + be_ref[...]
    o_ref[...] = jnp.maximum(y, 0.0)


_row_spec = pl.BlockSpec((_RB, D), lambda i: (i, 0))
_deg_spec = pl.BlockSpec((NC, _RB, DEGW), lambda i: (0, i, 0))
_part_spec = pl.BlockSpec((NC, _RB, D), lambda i: (0, i, 0))
_w_spec = pl.BlockSpec((D, D), lambda i: (0, 0))
_vec_spec = pl.BlockSpec((1, D), lambda i: (0, 0))

_GRID = (N // _RB,)

_premm = pl.pallas_call(
    _premm_body,
    grid=_GRID,
    in_specs=[_row_spec, _deg_spec, _w_spec],
    out_specs=_row_spec,
    out_shape=jax.ShapeDtypeStruct((N, D), jnp.float32),
)

_combmm = pl.pallas_call(
    _combmm_body,
    grid=_GRID,
    in_specs=[_part_spec, _row_spec, _deg_spec,
              _vec_spec, _vec_spec, _vec_spec, _w_spec],
    out_specs=_row_spec,
    out_shape=jax.ShapeDtypeStruct((N, D), jnp.float32),
)

_final = pl.pallas_call(
    _final_body,
    grid=_GRID,
    in_specs=[_part_spec, _row_spec, _deg_spec,
              _vec_spec, _vec_spec, _vec_spec],
    out_specs=_row_spec,
    out_shape=jax.ShapeDtypeStruct((N, D), jnp.float32),
)


def kernel(x, edge_index, W1, b1, g1, be1, W2, b2, g2, be2, W3, b3, g3, be3):
    src = edge_index[0].astype(jnp.int32)
    dst = edge_index[1].astype(jnp.int32)
    e = src.shape[0]
    nb = -(-e // (NC * NS * EB))
    nb = -(-nb // 8) * 8
    e_pad = NC * NS * nb * EB
    pad = e_pad - e
    srcp = jnp.concatenate([src, jnp.zeros((pad,), jnp.int32)])
    dstp = jnp.concatenate([dst, jnp.full((pad,), N, jnp.int32)])
    srcp = srcp.reshape(NC, NS, nb, EB)
    dstp = dstp.reshape(NC, NS, nb, EB)

    sc_deg, sc_scat = _make_sc_kernels(nb)

    degp = sc_deg(dstp)
    b1r, g1r, be1r = b1.reshape(1, D), g1.reshape(1, D), be1.reshape(1, D)
    b2r, g2r, be2r = b2.reshape(1, D), g2.reshape(1, D), be2.reshape(1, D)
    b3r, g3r, be3r = b3.reshape(1, D), g3.reshape(1, D), be3.reshape(1, D)

    hh1 = _premm(x, degp, W1)
    p1 = sc_scat(hh1, srcp, dstp)
    hh2 = _combmm(p1, hh1, degp, b1r, g1r, be1r, W2)
    p2 = sc_scat(hh2, srcp, dstp)
    hh3 = _combmm(p2, hh2, degp, b2r, g2r, be2r, W3)
    p3 = sc_scat(hh3, srcp, dstp)
    return _final(p3, hh3, degp, b3r, g3r, be3r)
